# Initial kernel scaffold; baseline (speedup 1.0000x reference)
import functools

import jax
import jax.numpy as jnp
from jax import lax
from jax.experimental import pallas as pl
from jax.experimental.pallas import tpu as pltpu
from jax.experimental.pallas import tpu_sc as plsc

N_NODES = 10000
N_EDGES = 320000
N_GRAPHS = 64
D = 128
D_OUT = 2

CHUNK = 128
ROWS = N_EDGES // CHUNK
NC = 2
NS = 16
NW = NC * NS
SLICE = N_NODES // NS

_mesh = plsc.VectorSubcoreMesh(core_axis_name="c", subcore_axis_name="s")


@functools.partial(
    pl.kernel,
    mesh=_mesh,
    out_type=(
        jax.ShapeDtypeStruct((NC, N_NODES, D), jnp.float32),
        jax.ShapeDtypeStruct((NC, N_NODES, 16), jnp.float32),
    ),
    scratch_types=(
        pltpu.VMEM((CHUNK,), jnp.int32),
        pltpu.VMEM((CHUNK,), jnp.int32),
        pltpu.VMEM((CHUNK, D), jnp.float32),
        pltpu.VMEM((CHUNK, 16), jnp.float32),
        pltpu.SemaphoreType.DMA,
        pltpu.VMEM_SHARED((N_NODES, D), jnp.float32),
        pltpu.VMEM_SHARED((N_NODES, 16), jnp.float32),
    ),
)
def _sc_agg_deg(x_hbm, src_hbm, dst_hbm, ones_hbm, z128_hbm, z16_hbm,
                acc_out, deg_out,
                src_v, dst_v, rows_v, ones_v, sem, acc_sh, deg_sh):
    c = lax.axis_index("c")
    s = lax.axis_index("s")
    wid = s * NC + c
    pltpu.sync_copy(z128_hbm, acc_sh.at[pl.ds(s * SLICE, SLICE)])
    pltpu.sync_copy(z16_hbm, deg_sh.at[pl.ds(s * SLICE, SLICE)])
    pltpu.sync_copy(ones_hbm, ones_v)
    plsc.subcore_barrier()

    nrows = (ROWS - wid + NW - 1) // NW

    def body(i, carry):
        j = wid + i * NW
        pltpu.sync_copy(src_hbm.at[j], src_v)
        pltpu.sync_copy(dst_hbm.at[j], dst_v)
        pltpu.async_copy(x_hbm.at[src_v], rows_v, sem).wait()
        pltpu.sync_copy(rows_v, acc_sh.at[dst_v], add=True)
        pltpu.sync_copy(ones_v, deg_sh.at[dst_v], add=True)
        return carry

    lax.fori_loop(0, nrows, body, 0)
    plsc.subcore_barrier()
    pltpu.sync_copy(acc_sh.at[pl.ds(s * SLICE, SLICE)],
                    acc_out.at[c, pl.ds(s * SLICE, SLICE)])
    pltpu.sync_copy(deg_sh.at[pl.ds(s * SLICE, SLICE)],
                    deg_out.at[c, pl.ds(s * SLICE, SLICE)])


@functools.partial(
    pl.kernel,
    mesh=_mesh,
    out_type=jax.ShapeDtypeStruct((NC, N_NODES, D), jnp.float32),
    scratch_types=(
        pltpu.VMEM((CHUNK,), jnp.int32),
        pltpu.VMEM((CHUNK,), jnp.int32),
        pltpu.VMEM((CHUNK, D), jnp.float32),
        pltpu.SemaphoreType.DMA,
        pltpu.VMEM_SHARED((N_NODES, D), jnp.float32),
    ),
)
def _sc_agg(x_hbm, src_hbm, dst_hbm, z128_hbm,
            acc_out,
            src_v, dst_v, rows_v, sem, acc_sh):
    c = lax.axis_index("c")
    s = lax.axis_index("s")
    wid = s * NC + c
    pltpu.sync_copy(z128_hbm, acc_sh.at[pl.ds(s * SLICE, SLICE)])
    plsc.subcore_barrier()

    nrows = (ROWS - wid + NW - 1) // NW

    def body(i, carry):
        j = wid + i * NW
        pltpu.sync_copy(src_hbm.at[j], src_v)
        pltpu.sync_copy(dst_hbm.at[j], dst_v)
        pltpu.async_copy(x_hbm.at[src_v], rows_v, sem).wait()
        pltpu.sync_copy(rows_v, acc_sh.at[dst_v], add=True)
        return carry

    lax.fori_loop(0, nrows, body, 0)
    plsc.subcore_barrier()
    pltpu.sync_copy(acc_sh.at[pl.ds(s * SLICE, SLICE)],
                    acc_out.at[c, pl.ds(s * SLICE, SLICE)])


BLK = 1000
GRID = N_NODES // BLK


def _tc_layer1_body(acc_ref, deg_ref, x_ref, wl_ref, wr_ref, bl_ref,
                    h_ref, invd_ref):
    acc = acc_ref[0] + acc_ref[1]
    deg = deg_ref[0, :, 0:1] + deg_ref[1, :, 0:1]
    invd = 1.0 / jnp.maximum(deg, 1.0)
    h = jnp.dot(acc * invd, wl_ref[...], preferred_element_type=jnp.float32)
    h = h + jnp.dot(x_ref[...], wr_ref[...], preferred_element_type=jnp.float32)
    h = h + bl_ref[...]
    h_ref[...] = jnp.maximum(h, 0.0)
    invd_ref[...] = jnp.broadcast_to(invd, (BLK, 16))


def _tc_layer2_body(acc_ref, invd_ref, h1_ref, batch_ref, wl_ref, wr_ref,
                    bl_ref, wlin_ref, blin_ref, out_ref, pooled_acc, cnt_acc):
    i = pl.program_id(0)

    @pl.when(i == 0)
    def _():
        pooled_acc[...] = jnp.zeros_like(pooled_acc)
        cnt_acc[...] = jnp.zeros_like(cnt_acc)

    acc = acc_ref[0] + acc_ref[1]
    invd = invd_ref[:, 0:1]
    h2 = jnp.dot(acc * invd, wl_ref[...], preferred_element_type=jnp.float32)
    h2 = h2 + jnp.dot(h1_ref[...], wr_ref[...],
                      preferred_element_type=jnp.float32)
    h2 = jnp.maximum(h2 + bl_ref[...], 0.0)

    b = batch_ref[0]
    gids = lax.broadcasted_iota(jnp.int32, (N_GRAPHS, 1), 0)
    onehot = (gids == b).astype(jnp.float32)
    pooled_acc[...] += jnp.dot(onehot, h2, preferred_element_type=jnp.float32)
    cnt_acc[...] += jnp.broadcast_to(
        jnp.sum(onehot, axis=1, keepdims=True), (N_GRAPHS, D))

    @pl.when(i == GRID - 1)
    def _():
        pooled = pooled_acc[...] / jnp.maximum(cnt_acc[...], 1.0)
        out_ref[...] = jnp.dot(pooled, wlin_ref[...],
                               preferred_element_type=jnp.float32) + blin_ref[...]


def kernel(x, edge_index, batch, Wl1, bl1, Wr1, Wl2, bl2, Wr2, Wlin, blin):
    src2d = edge_index[0].reshape(ROWS, CHUNK)
    dst2d = edge_index[1].reshape(ROWS, CHUNK)
    ones16 = jnp.ones((CHUNK, 16), jnp.float32)
    z128 = jnp.zeros((SLICE, D), jnp.float32)
    z16 = jnp.zeros((SLICE, 16), jnp.float32)

    acc1, deg1 = _sc_agg_deg(x, src2d, dst2d, ones16, z128, z16)

    h1, invd16 = pl.pallas_call(
        _tc_layer1_body,
        grid=(GRID,),
        in_specs=[
            pl.BlockSpec((NC, BLK, D), lambda i: (0, i, 0)),
            pl.BlockSpec((NC, BLK, 16), lambda i: (0, i, 0)),
            pl.BlockSpec((BLK, D), lambda i: (i, 0)),
            pl.BlockSpec((D, D), lambda i: (0, 0)),
            pl.BlockSpec((D, D), lambda i: (0, 0)),
            pl.BlockSpec((1, D), lambda i: (0, 0)),
        ],
        out_specs=[
            pl.BlockSpec((BLK, D), lambda i: (i, 0)),
            pl.BlockSpec((BLK, 16), lambda i: (i, 0)),
        ],
        out_shape=[
            jax.ShapeDtypeStruct((N_NODES, D), jnp.float32),
            jax.ShapeDtypeStruct((N_NODES, 16), jnp.float32),
        ],
    )(acc1, deg1, x, Wl1.T, Wr1.T, bl1[None, :])

    acc2 = _sc_agg(h1, src2d, dst2d, z128)

    wlin_pad = jnp.pad(Wlin.T, ((0, 0), (0, D - D_OUT)))
    blin_pad = jnp.pad(blin[None, :], ((0, 0), (0, D - D_OUT)))
    batch3d = batch.reshape(GRID, 1, BLK)

    out_pad = pl.pallas_call(
        _tc_layer2_body,
        grid=(GRID,),
        in_specs=[
            pl.BlockSpec((NC, BLK, D), lambda i: (0, i, 0)),
            pl.BlockSpec((BLK, 16), lambda i: (i, 0)),
            pl.BlockSpec((BLK, D), lambda i: (i, 0)),
            pl.BlockSpec((1, 1, BLK), lambda i: (i, 0, 0)),
            pl.BlockSpec((D, D), lambda i: (0, 0)),
            pl.BlockSpec((D, D), lambda i: (0, 0)),
            pl.BlockSpec((1, D), lambda i: (0, 0)),
            pl.BlockSpec((D, D), lambda i: (0, 0)),
            pl.BlockSpec((1, D), lambda i: (0, 0)),
        ],
        out_specs=pl.BlockSpec((N_GRAPHS, D), lambda i: (0, 0)),
        out_shape=jax.ShapeDtypeStruct((N_GRAPHS, D), jnp.float32),
        scratch_shapes=[
            pltpu.VMEM((N_GRAPHS, D), jnp.float32),
            pltpu.VMEM((N_GRAPHS, D), jnp.float32),
        ],
    )(acc2, invd16, h1, batch3d, Wl2.T, Wr2.T, bl2[None, :],
      wlin_pad, blin_pad)

    return out_pad[:, :D_OUT]


# speedup vs baseline: 6.7990x; 6.7990x over previous
import functools

import jax
import jax.numpy as jnp
from jax import lax
from jax.experimental import pallas as pl
from jax.experimental.pallas import tpu as pltpu
from jax.experimental.pallas import tpu_sc as plsc

N_NODES = 10000
N_EDGES = 320000
N_GRAPHS = 64
D = 128
D_OUT = 2

CHUNK = 128
ROWS = N_EDGES // CHUNK
NC = 2
NS = 16
NW = NC * NS
SLICE = 632

_mesh = plsc.VectorSubcoreMesh(core_axis_name="c", subcore_axis_name="s")
_sc_params = pltpu.CompilerParams(use_tc_tiling_on_sc=False)


def _sc_agg_deg_body(x_hbm, src_hbm, dst_hbm, ones_hbm, z128_hbm, z16_hbm,
                acc_out, deg_out,
                src_v, dst_v, rows_v, ones_v, sem, acc_sh, deg_sh):
    c = lax.axis_index("c")
    s = lax.axis_index("s")
    wid = s * NC + c
    start = lax.min(s * SLICE, N_NODES - SLICE)
    pltpu.sync_copy(z128_hbm, rows_v)
    pltpu.sync_copy(z16_hbm, ones_v)
    for k in range(4):
        pltpu.sync_copy(rows_v, acc_sh.at[pl.ds(start + k * CHUNK, CHUNK)])
        pltpu.sync_copy(ones_v, deg_sh.at[pl.ds(start + k * CHUNK, CHUNK)])
    pltpu.sync_copy(rows_v.at[pl.ds(0, 120)],
                    acc_sh.at[pl.ds(start + 4 * CHUNK, 120)])
    pltpu.sync_copy(ones_v.at[pl.ds(0, 120)],
                    deg_sh.at[pl.ds(start + 4 * CHUNK, 120)])
    pltpu.sync_copy(ones_hbm, ones_v)
    plsc.subcore_barrier()

    nrows = (ROWS - wid + NW - 1) // NW

    def body(i, carry):
        j = wid + i * NW
        pltpu.sync_copy(src_hbm.at[j], src_v)
        pltpu.sync_copy(dst_hbm.at[j], dst_v)
        pltpu.async_copy(x_hbm.at[src_v], rows_v, sem).wait()
        pltpu.sync_copy(rows_v, acc_sh.at[dst_v], add=True)
        pltpu.sync_copy(ones_v, deg_sh.at[dst_v], add=True)
        return carry

    lax.fori_loop(0, nrows, body, 0)
    plsc.subcore_barrier()
    for k in range(4):
        pltpu.sync_copy(acc_sh.at[pl.ds(start + k * CHUNK, CHUNK)], rows_v)
        pltpu.sync_copy(rows_v, acc_out.at[c, pl.ds(start + k * CHUNK, CHUNK)])
        pltpu.sync_copy(deg_sh.at[pl.ds(start + k * CHUNK, CHUNK)], ones_v)
        pltpu.sync_copy(ones_v, deg_out.at[c, pl.ds(start + k * CHUNK, CHUNK)])
    pltpu.sync_copy(acc_sh.at[pl.ds(start + 4 * CHUNK, 120)],
                    rows_v.at[pl.ds(0, 120)])
    pltpu.sync_copy(rows_v.at[pl.ds(0, 120)],
                    acc_out.at[c, pl.ds(start + 4 * CHUNK, 120)])
    pltpu.sync_copy(deg_sh.at[pl.ds(start + 4 * CHUNK, 120)],
                    ones_v.at[pl.ds(0, 120)])
    pltpu.sync_copy(ones_v.at[pl.ds(0, 120)],
                    deg_out.at[c, pl.ds(start + 4 * CHUNK, 120)])


def _sc_agg_body(x_hbm, src_hbm, dst_hbm, z128_hbm,
            acc_out,
            src_v, dst_v, rows_v, sem, acc_sh):
    c = lax.axis_index("c")
    s = lax.axis_index("s")
    wid = s * NC + c
    start = lax.min(s * SLICE, N_NODES - SLICE)
    pltpu.sync_copy(z128_hbm, rows_v)
    for k in range(4):
        pltpu.sync_copy(rows_v, acc_sh.at[pl.ds(start + k * CHUNK, CHUNK)])
    pltpu.sync_copy(rows_v.at[pl.ds(0, 120)],
                    acc_sh.at[pl.ds(start + 4 * CHUNK, 120)])
    plsc.subcore_barrier()

    nrows = (ROWS - wid + NW - 1) // NW

    def body(i, carry):
        j = wid + i * NW
        pltpu.sync_copy(src_hbm.at[j], src_v)
        pltpu.sync_copy(dst_hbm.at[j], dst_v)
        pltpu.async_copy(x_hbm.at[src_v], rows_v, sem).wait()
        pltpu.sync_copy(rows_v, acc_sh.at[dst_v], add=True)
        return carry

    lax.fori_loop(0, nrows, body, 0)
    plsc.subcore_barrier()
    for k in range(4):
        pltpu.sync_copy(acc_sh.at[pl.ds(start + k * CHUNK, CHUNK)], rows_v)
        pltpu.sync_copy(rows_v, acc_out.at[c, pl.ds(start + k * CHUNK, CHUNK)])
    pltpu.sync_copy(acc_sh.at[pl.ds(start + 4 * CHUNK, 120)],
                    rows_v.at[pl.ds(0, 120)])
    pltpu.sync_copy(rows_v.at[pl.ds(0, 120)],
                    acc_out.at[c, pl.ds(start + 4 * CHUNK, 120)])


def _make_sc_kernels(interpret=False):
    agg_deg = pl.kernel(
        _sc_agg_deg_body,
        mesh=_mesh,
        compiler_params=_sc_params,
        out_type=(
            jax.ShapeDtypeStruct((NC, N_NODES, D), jnp.float32),
            jax.ShapeDtypeStruct((NC, N_NODES, 16), jnp.float32),
        ),
        scratch_types=(
            pltpu.VMEM((CHUNK,), jnp.int32),
            pltpu.VMEM((CHUNK,), jnp.int32),
            pltpu.VMEM((CHUNK, D), jnp.float32),
            pltpu.VMEM((CHUNK, 16), jnp.float32),
            pltpu.SemaphoreType.DMA,
            pltpu.VMEM_SHARED((N_NODES, D), jnp.float32),
            pltpu.VMEM_SHARED((N_NODES, 16), jnp.float32),
        ),
        interpret=interpret,
    )
    agg = pl.kernel(
        _sc_agg_body,
        mesh=_mesh,
        compiler_params=_sc_params,
        out_type=jax.ShapeDtypeStruct((NC, N_NODES, D), jnp.float32),
        scratch_types=(
            pltpu.VMEM((CHUNK,), jnp.int32),
            pltpu.VMEM((CHUNK,), jnp.int32),
            pltpu.VMEM((CHUNK, D), jnp.float32),
            pltpu.SemaphoreType.DMA,
            pltpu.VMEM_SHARED((N_NODES, D), jnp.float32),
        ),
        interpret=interpret,
    )
    return agg_deg, agg


_sc_agg_deg, _sc_agg = _make_sc_kernels()


BLK = 1000
GRID = N_NODES // BLK


def _tc_layer1_body(acc_ref, deg_ref, x_ref, wl_ref, wr_ref, bl_ref,
                    h_ref, invd_ref):
    acc = acc_ref[0] + acc_ref[1]
    deg = deg_ref[0, :, 0:1] + deg_ref[1, :, 0:1]
    invd = 1.0 / jnp.maximum(deg, 1.0)
    h = jnp.dot(acc * invd, wl_ref[...], preferred_element_type=jnp.float32)
    h = h + jnp.dot(x_ref[...], wr_ref[...], preferred_element_type=jnp.float32)
    h = h + bl_ref[...]
    h_ref[...] = jnp.maximum(h, 0.0)
    invd_ref[...] = jnp.broadcast_to(invd, (BLK, 16))


def _tc_layer2_body(acc_ref, invd_ref, h1_ref, batch_ref, wl_ref, wr_ref,
                    bl_ref, wlin_ref, blin_ref, out_ref, pooled_acc, cnt_acc):
    i = pl.program_id(0)

    @pl.when(i == 0)
    def _():
        pooled_acc[...] = jnp.zeros_like(pooled_acc)
        cnt_acc[...] = jnp.zeros_like(cnt_acc)

    acc = acc_ref[0] + acc_ref[1]
    invd = invd_ref[:, 0:1]
    h2 = jnp.dot(acc * invd, wl_ref[...], preferred_element_type=jnp.float32)
    h2 = h2 + jnp.dot(h1_ref[...], wr_ref[...],
                      preferred_element_type=jnp.float32)
    h2 = jnp.maximum(h2 + bl_ref[...], 0.0)

    b = batch_ref[0]
    gids = lax.broadcasted_iota(jnp.int32, (N_GRAPHS, 1), 0)
    onehot = (gids == b).astype(jnp.float32)
    pooled_acc[...] += jnp.dot(onehot, h2, preferred_element_type=jnp.float32)
    cnt_acc[...] += jnp.broadcast_to(
        jnp.sum(onehot, axis=1, keepdims=True), (N_GRAPHS, D))

    @pl.when(i == GRID - 1)
    def _():
        pooled = pooled_acc[...] / jnp.maximum(cnt_acc[...], 1.0)
        out_ref[...] = jnp.dot(pooled, wlin_ref[...],
                               preferred_element_type=jnp.float32) + blin_ref[...]


def kernel(x, edge_index, batch, Wl1, bl1, Wr1, Wl2, bl2, Wr2, Wlin, blin):
    src2d = edge_index[0].reshape(ROWS, CHUNK)
    dst2d = edge_index[1].reshape(ROWS, CHUNK)
    ones16 = jnp.ones((CHUNK, 16), jnp.float32)
    z128 = jnp.zeros((CHUNK, D), jnp.float32)
    z16 = jnp.zeros((CHUNK, 16), jnp.float32)

    acc1, deg1 = _sc_agg_deg(x, src2d, dst2d, ones16, z128, z16)

    h1, invd16 = pl.pallas_call(
        _tc_layer1_body,
        grid=(GRID,),
        in_specs=[
            pl.BlockSpec((NC, BLK, D), lambda i: (0, i, 0)),
            pl.BlockSpec((NC, BLK, 16), lambda i: (0, i, 0)),
            pl.BlockSpec((BLK, D), lambda i: (i, 0)),
            pl.BlockSpec((D, D), lambda i: (0, 0)),
            pl.BlockSpec((D, D), lambda i: (0, 0)),
            pl.BlockSpec((1, D), lambda i: (0, 0)),
        ],
        out_specs=[
            pl.BlockSpec((BLK, D), lambda i: (i, 0)),
            pl.BlockSpec((BLK, 16), lambda i: (i, 0)),
        ],
        out_shape=[
            jax.ShapeDtypeStruct((N_NODES, D), jnp.float32),
            jax.ShapeDtypeStruct((N_NODES, 16), jnp.float32),
        ],
    )(acc1, deg1, x, Wl1.T, Wr1.T, bl1[None, :])

    acc2 = _sc_agg(h1, src2d, dst2d, z128)

    wlin_pad = jnp.pad(Wlin.T, ((0, 0), (0, D - D_OUT)))
    blin_pad = jnp.pad(blin[None, :], ((0, 0), (0, D - D_OUT)))
    batch3d = batch.reshape(GRID, 1, BLK)

    out_pad = pl.pallas_call(
        _tc_layer2_body,
        grid=(GRID,),
        in_specs=[
            pl.BlockSpec((NC, BLK, D), lambda i: (0, i, 0)),
            pl.BlockSpec((BLK, 16), lambda i: (i, 0)),
            pl.BlockSpec((BLK, D), lambda i: (i, 0)),
            pl.BlockSpec((1, 1, BLK), lambda i: (i, 0, 0)),
            pl.BlockSpec((D, D), lambda i: (0, 0)),
            pl.BlockSpec((D, D), lambda i: (0, 0)),
            pl.BlockSpec((1, D), lambda i: (0, 0)),
            pl.BlockSpec((D, D), lambda i: (0, 0)),
            pl.BlockSpec((1, D), lambda i: (0, 0)),
        ],
        out_specs=pl.BlockSpec((N_GRAPHS, D), lambda i: (0, 0)),
        out_shape=jax.ShapeDtypeStruct((N_GRAPHS, D), jnp.float32),
        scratch_shapes=[
            pltpu.VMEM((N_GRAPHS, D), jnp.float32),
            pltpu.VMEM((N_GRAPHS, D), jnp.float32),
        ],
    )(acc2, invd16, h1, batch3d, Wl2.T, Wr2.T, bl2[None, :],
      wlin_pad, blin_pad)

    return out_pad[:, :D_OUT]


# speedup vs baseline: 10.3238x; 1.5184x over previous
import functools

import jax
import jax.numpy as jnp
from jax import lax
from jax.experimental import pallas as pl
from jax.experimental.pallas import tpu as pltpu
from jax.experimental.pallas import tpu_sc as plsc

N_NODES = 10000
N_EDGES = 320000
N_GRAPHS = 64
D = 128
D_OUT = 2

CHUNK = 128
ROWS = N_EDGES // CHUNK
NC = 2
NS = 16
NW = NC * NS
SLICE = 632

_mesh = plsc.VectorSubcoreMesh(core_axis_name="c", subcore_axis_name="s")
_sc_params = pltpu.CompilerParams(use_tc_tiling_on_sc=False)


NBUF = 2


def _sc_agg_deg_body(x_hbm, src_hbm, dst_hbm, ones_hbm, z128_hbm, z16_hbm,
                acc_out, deg_out,
                src_v, dst_v, rows_v, ones_v,
                sem0, sem1, acc_sh, deg_sh):
    c = lax.axis_index("c")
    s = lax.axis_index("s")
    wid = s * NC + c
    start = lax.min(s * SLICE, N_NODES - SLICE)
    pltpu.sync_copy(z128_hbm, rows_v.at[0])
    pltpu.sync_copy(z16_hbm, ones_v)
    for k in range(4):
        pltpu.sync_copy(rows_v.at[0], acc_sh.at[pl.ds(start + k * CHUNK, CHUNK)])
        pltpu.sync_copy(ones_v, deg_sh.at[pl.ds(start + k * CHUNK, CHUNK)])
    pltpu.sync_copy(rows_v.at[0, pl.ds(0, 120)],
                    acc_sh.at[pl.ds(start + 4 * CHUNK, 120)])
    pltpu.sync_copy(ones_v.at[pl.ds(0, 120)],
                    deg_sh.at[pl.ds(start + 4 * CHUNK, 120)])
    pltpu.sync_copy(ones_hbm, ones_v)
    plsc.subcore_barrier()

    nrows = (ROWS - wid + NW - 1) // NW
    sems = (sem0, sem1)

    def fire(b, i):
        j = wid + i * NW
        pltpu.sync_copy(src_hbm.at[j], src_v.at[b])
        pltpu.sync_copy(dst_hbm.at[j], dst_v.at[b])
        pltpu.async_copy(x_hbm.at[src_v.at[b]], rows_v.at[b], sems[b])

    for b in range(NBUF):
        fire(b, b)

    def body(k, carry):
        for b in range(NBUF):
            i = k * NBUF + b

            @pl.when(i < nrows)
            def _():
                pltpu.make_async_copy(x_hbm.at[src_v.at[b]], rows_v.at[b],
                                      sems[b]).wait()
                pltpu.sync_copy(rows_v.at[b], acc_sh.at[dst_v.at[b]], add=True)
                pltpu.sync_copy(ones_v, deg_sh.at[dst_v.at[b]], add=True)

                @pl.when(i + NBUF < nrows)
                def _():
                    fire(b, i + NBUF)
        return carry

    lax.fori_loop(0, (nrows + NBUF - 1) // NBUF, body, 0)
    plsc.subcore_barrier()
    for k in range(4):
        pltpu.sync_copy(acc_sh.at[pl.ds(start + k * CHUNK, CHUNK)], rows_v.at[0])
        pltpu.sync_copy(rows_v.at[0],
                        acc_out.at[c, pl.ds(start + k * CHUNK, CHUNK)])
        pltpu.sync_copy(deg_sh.at[pl.ds(start + k * CHUNK, CHUNK)], ones_v)
        pltpu.sync_copy(ones_v, deg_out.at[c, pl.ds(start + k * CHUNK, CHUNK)])
    pltpu.sync_copy(acc_sh.at[pl.ds(start + 4 * CHUNK, 120)],
                    rows_v.at[0, pl.ds(0, 120)])
    pltpu.sync_copy(rows_v.at[0, pl.ds(0, 120)],
                    acc_out.at[c, pl.ds(start + 4 * CHUNK, 120)])
    pltpu.sync_copy(deg_sh.at[pl.ds(start + 4 * CHUNK, 120)],
                    ones_v.at[pl.ds(0, 120)])
    pltpu.sync_copy(ones_v.at[pl.ds(0, 120)],
                    deg_out.at[c, pl.ds(start + 4 * CHUNK, 120)])


def _sc_agg_body(x_hbm, src_hbm, dst_hbm, z128_hbm,
            acc_out,
            src_v, dst_v, rows_v, sem0, sem1, acc_sh):
    c = lax.axis_index("c")
    s = lax.axis_index("s")
    wid = s * NC + c
    start = lax.min(s * SLICE, N_NODES - SLICE)
    pltpu.sync_copy(z128_hbm, rows_v.at[0])
    for k in range(4):
        pltpu.sync_copy(rows_v.at[0], acc_sh.at[pl.ds(start + k * CHUNK, CHUNK)])
    pltpu.sync_copy(rows_v.at[0, pl.ds(0, 120)],
                    acc_sh.at[pl.ds(start + 4 * CHUNK, 120)])
    plsc.subcore_barrier()

    nrows = (ROWS - wid + NW - 1) // NW
    sems = (sem0, sem1)

    def fire(b, i):
        j = wid + i * NW
        pltpu.sync_copy(src_hbm.at[j], src_v.at[b])
        pltpu.sync_copy(dst_hbm.at[j], dst_v.at[b])
        pltpu.async_copy(x_hbm.at[src_v.at[b]], rows_v.at[b], sems[b])

    for b in range(NBUF):
        fire(b, b)

    def body(k, carry):
        for b in range(NBUF):
            i = k * NBUF + b

            @pl.when(i < nrows)
            def _():
                pltpu.make_async_copy(x_hbm.at[src_v.at[b]], rows_v.at[b],
                                      sems[b]).wait()
                pltpu.sync_copy(rows_v.at[b], acc_sh.at[dst_v.at[b]], add=True)

                @pl.when(i + NBUF < nrows)
                def _():
                    fire(b, i + NBUF)
        return carry

    lax.fori_loop(0, (nrows + NBUF - 1) // NBUF, body, 0)
    plsc.subcore_barrier()
    for k in range(4):
        pltpu.sync_copy(acc_sh.at[pl.ds(start + k * CHUNK, CHUNK)], rows_v.at[0])
        pltpu.sync_copy(rows_v.at[0],
                        acc_out.at[c, pl.ds(start + k * CHUNK, CHUNK)])
    pltpu.sync_copy(acc_sh.at[pl.ds(start + 4 * CHUNK, 120)],
                    rows_v.at[0, pl.ds(0, 120)])
    pltpu.sync_copy(rows_v.at[0, pl.ds(0, 120)],
                    acc_out.at[c, pl.ds(start + 4 * CHUNK, 120)])


def _make_sc_kernels(interpret=False):
    agg_deg = pl.kernel(
        _sc_agg_deg_body,
        mesh=_mesh,
        compiler_params=_sc_params,
        out_type=(
            jax.ShapeDtypeStruct((NC, N_NODES, D), jnp.float32),
            jax.ShapeDtypeStruct((NC, N_NODES, 16), jnp.float32),
        ),
        scratch_types=(
            pltpu.VMEM((NBUF, CHUNK), jnp.int32),
            pltpu.VMEM((NBUF, CHUNK), jnp.int32),
            pltpu.VMEM((NBUF, CHUNK, D), jnp.float32),
            pltpu.VMEM((CHUNK, 16), jnp.float32),
            pltpu.SemaphoreType.DMA,
            pltpu.SemaphoreType.DMA,
            pltpu.VMEM_SHARED((N_NODES, D), jnp.float32),
            pltpu.VMEM_SHARED((N_NODES, 16), jnp.float32),
        ),
        interpret=interpret,
    )
    agg = pl.kernel(
        _sc_agg_body,
        mesh=_mesh,
        compiler_params=_sc_params,
        out_type=jax.ShapeDtypeStruct((NC, N_NODES, D), jnp.float32),
        scratch_types=(
            pltpu.VMEM((NBUF, CHUNK), jnp.int32),
            pltpu.VMEM((NBUF, CHUNK), jnp.int32),
            pltpu.VMEM((NBUF, CHUNK, D), jnp.float32),
            pltpu.SemaphoreType.DMA,
            pltpu.SemaphoreType.DMA,
            pltpu.VMEM_SHARED((N_NODES, D), jnp.float32),
        ),
        interpret=interpret,
    )
    return agg_deg, agg


_sc_agg_deg, _sc_agg = _make_sc_kernels()


BLK = 1000
GRID = N_NODES // BLK


def _tc_layer1_body(acc_ref, deg_ref, x_ref, wl_ref, wr_ref, bl_ref,
                    h_ref, invd_ref):
    acc = acc_ref[0] + acc_ref[1]
    deg = deg_ref[0, :, 0:1] + deg_ref[1, :, 0:1]
    invd = 1.0 / jnp.maximum(deg, 1.0)
    h = jnp.dot(acc * invd, wl_ref[...], preferred_element_type=jnp.float32)
    h = h + jnp.dot(x_ref[...], wr_ref[...], preferred_element_type=jnp.float32)
    h = h + bl_ref[...]
    h_ref[...] = jnp.maximum(h, 0.0)
    invd_ref[...] = jnp.broadcast_to(invd, (BLK, 16))


def _tc_layer2_body(acc_ref, invd_ref, h1_ref, batch_ref, wl_ref, wr_ref,
                    bl_ref, wlin_ref, blin_ref, out_ref, pooled_acc, cnt_acc):
    i = pl.program_id(0)

    @pl.when(i == 0)
    def _():
        pooled_acc[...] = jnp.zeros_like(pooled_acc)
        cnt_acc[...] = jnp.zeros_like(cnt_acc)

    acc = acc_ref[0] + acc_ref[1]
    invd = invd_ref[:, 0:1]
    h2 = jnp.dot(acc * invd, wl_ref[...], preferred_element_type=jnp.float32)
    h2 = h2 + jnp.dot(h1_ref[...], wr_ref[...],
                      preferred_element_type=jnp.float32)
    h2 = jnp.maximum(h2 + bl_ref[...], 0.0)

    b = batch_ref[0]
    gids = lax.broadcasted_iota(jnp.int32, (N_GRAPHS, 1), 0)
    onehot = (gids == b).astype(jnp.float32)
    pooled_acc[...] += jnp.dot(onehot, h2, preferred_element_type=jnp.float32)
    cnt_acc[...] += jnp.broadcast_to(
        jnp.sum(onehot, axis=1, keepdims=True), (N_GRAPHS, D))

    @pl.when(i == GRID - 1)
    def _():
        pooled = pooled_acc[...] / jnp.maximum(cnt_acc[...], 1.0)
        out_ref[...] = jnp.dot(pooled, wlin_ref[...],
                               preferred_element_type=jnp.float32) + blin_ref[...]


def kernel(x, edge_index, batch, Wl1, bl1, Wr1, Wl2, bl2, Wr2, Wlin, blin):
    src2d = edge_index[0].reshape(ROWS, CHUNK)
    dst2d = edge_index[1].reshape(ROWS, CHUNK)
    ones16 = jnp.ones((CHUNK, 16), jnp.float32)
    z128 = jnp.zeros((CHUNK, D), jnp.float32)
    z16 = jnp.zeros((CHUNK, 16), jnp.float32)

    acc1, deg1 = _sc_agg_deg(x, src2d, dst2d, ones16, z128, z16)

    h1, invd16 = pl.pallas_call(
        _tc_layer1_body,
        grid=(GRID,),
        in_specs=[
            pl.BlockSpec((NC, BLK, D), lambda i: (0, i, 0)),
            pl.BlockSpec((NC, BLK, 16), lambda i: (0, i, 0)),
            pl.BlockSpec((BLK, D), lambda i: (i, 0)),
            pl.BlockSpec((D, D), lambda i: (0, 0)),
            pl.BlockSpec((D, D), lambda i: (0, 0)),
            pl.BlockSpec((1, D), lambda i: (0, 0)),
        ],
        out_specs=[
            pl.BlockSpec((BLK, D), lambda i: (i, 0)),
            pl.BlockSpec((BLK, 16), lambda i: (i, 0)),
        ],
        out_shape=[
            jax.ShapeDtypeStruct((N_NODES, D), jnp.float32),
            jax.ShapeDtypeStruct((N_NODES, 16), jnp.float32),
        ],
    )(acc1, deg1, x, Wl1.T, Wr1.T, bl1[None, :])

    acc2 = _sc_agg(h1, src2d, dst2d, z128)

    wlin_pad = jnp.pad(Wlin.T, ((0, 0), (0, D - D_OUT)))
    blin_pad = jnp.pad(blin[None, :], ((0, 0), (0, D - D_OUT)))
    batch3d = batch.reshape(GRID, 1, BLK)

    out_pad = pl.pallas_call(
        _tc_layer2_body,
        grid=(GRID,),
        in_specs=[
            pl.BlockSpec((NC, BLK, D), lambda i: (0, i, 0)),
            pl.BlockSpec((BLK, 16), lambda i: (i, 0)),
            pl.BlockSpec((BLK, D), lambda i: (i, 0)),
            pl.BlockSpec((1, 1, BLK), lambda i: (i, 0, 0)),
            pl.BlockSpec((D, D), lambda i: (0, 0)),
            pl.BlockSpec((D, D), lambda i: (0, 0)),
            pl.BlockSpec((1, D), lambda i: (0, 0)),
            pl.BlockSpec((D, D), lambda i: (0, 0)),
            pl.BlockSpec((1, D), lambda i: (0, 0)),
        ],
        out_specs=pl.BlockSpec((N_GRAPHS, D), lambda i: (0, 0)),
        out_shape=jax.ShapeDtypeStruct((N_GRAPHS, D), jnp.float32),
        scratch_shapes=[
            pltpu.VMEM((N_GRAPHS, D), jnp.float32),
            pltpu.VMEM((N_GRAPHS, D), jnp.float32),
        ],
    )(acc2, invd16, h1, batch3d, Wl2.T, Wr2.T, bl2[None, :],
      wlin_pad, blin_pad)

    return out_pad[:, :D_OUT]


# speedup vs baseline: 10.4600x; 1.0132x over previous
import functools

import jax
import jax.numpy as jnp
from jax import lax
from jax.experimental import pallas as pl
from jax.experimental.pallas import tpu as pltpu
from jax.experimental.pallas import tpu_sc as plsc

N_NODES = 10000
N_EDGES = 320000
N_GRAPHS = 64
D = 128
D_OUT = 2

CHUNK = 128
ROWS = N_EDGES // CHUNK
NC = 2
NS = 16
NW = NC * NS
SLICE = 632

_mesh = plsc.VectorSubcoreMesh(core_axis_name="c", subcore_axis_name="s")
_sc_params = pltpu.CompilerParams(use_tc_tiling_on_sc=False)


NBUF = 2


def _sc_agg_deg_body(x_hbm, src_hbm, dst_hbm, ones_hbm, z128_hbm, z16_hbm,
                acc_out, deg_out,
                src_v, dst_v, rows_v, ones_v,
                sem0, sem1, acc_sh, deg_sh):
    c = lax.axis_index("c")
    s = lax.axis_index("s")
    wid = s * NC + c
    start = lax.min(s * SLICE, N_NODES - SLICE)
    pltpu.sync_copy(z128_hbm, rows_v.at[0])
    pltpu.sync_copy(z16_hbm, ones_v)
    for k in range(4):
        pltpu.async_copy(rows_v.at[0],
                         acc_sh.at[pl.ds(start + k * CHUNK, CHUNK)], sem0)
        pltpu.async_copy(ones_v,
                         deg_sh.at[pl.ds(start + k * CHUNK, CHUNK)], sem1)
    pltpu.async_copy(rows_v.at[0, pl.ds(0, 120)],
                     acc_sh.at[pl.ds(start + 4 * CHUNK, 120)], sem0)
    pltpu.async_copy(ones_v.at[pl.ds(0, 120)],
                     deg_sh.at[pl.ds(start + 4 * CHUNK, 120)], sem1)
    for k in range(4):
        pltpu.make_async_copy(
            rows_v.at[0], acc_sh.at[pl.ds(start + k * CHUNK, CHUNK)],
            sem0).wait()
        pltpu.make_async_copy(
            ones_v, deg_sh.at[pl.ds(start + k * CHUNK, CHUNK)], sem1).wait()
    pltpu.make_async_copy(
        rows_v.at[0, pl.ds(0, 120)],
        acc_sh.at[pl.ds(start + 4 * CHUNK, 120)], sem0).wait()
    pltpu.make_async_copy(
        ones_v.at[pl.ds(0, 120)],
        deg_sh.at[pl.ds(start + 4 * CHUNK, 120)], sem1).wait()
    pltpu.sync_copy(ones_hbm, ones_v)
    plsc.subcore_barrier()

    nrows = (ROWS - wid + NW - 1) // NW
    sems = (sem0, sem1)

    def fire(b, i):
        j = wid + i * NW
        pltpu.sync_copy(src_hbm.at[j], src_v.at[b])
        pltpu.sync_copy(dst_hbm.at[j], dst_v.at[b])
        pltpu.async_copy(x_hbm.at[src_v.at[b]], rows_v.at[b], sems[b])

    for b in range(NBUF):
        fire(b, b)

    def body(k, carry):
        for b in range(NBUF):
            i = k * NBUF + b

            @pl.when(i < nrows)
            def _():
                pltpu.make_async_copy(x_hbm.at[src_v.at[b]], rows_v.at[b],
                                      sems[b]).wait()
                pltpu.sync_copy(rows_v.at[b], acc_sh.at[dst_v.at[b]], add=True)
                pltpu.sync_copy(ones_v, deg_sh.at[dst_v.at[b]], add=True)

                @pl.when(i + NBUF < nrows)
                def _():
                    fire(b, i + NBUF)
        return carry

    lax.fori_loop(0, (nrows + NBUF - 1) // NBUF, body, 0)
    plsc.subcore_barrier()
    sizes = (CHUNK, CHUNK, CHUNK, CHUNK, 120)

    def stage(b, k, n):
        pltpu.sync_copy(acc_sh.at[pl.ds(start + k * CHUNK, n)],
                        rows_v.at[b, pl.ds(0, n)])
        pltpu.async_copy(rows_v.at[b, pl.ds(0, n)],
                         acc_out.at[c, pl.ds(start + k * CHUNK, n)], sems[b])

    def drain(b, k, n):
        pltpu.make_async_copy(
            rows_v.at[b, pl.ds(0, n)],
            acc_out.at[c, pl.ds(start + k * CHUNK, n)], sems[b]).wait()

    for k, n in enumerate(sizes):
        b = k % NBUF
        if k >= NBUF:
            drain(b, k - NBUF, sizes[k - NBUF])
        stage(b, k, n)
    for k in range(max(0, len(sizes) - NBUF), len(sizes)):
        drain(k % NBUF, k, sizes[k])
    for k in range(4):
        pltpu.sync_copy(deg_sh.at[pl.ds(start + k * CHUNK, CHUNK)], ones_v)
        pltpu.sync_copy(ones_v, deg_out.at[c, pl.ds(start + k * CHUNK, CHUNK)])
    pltpu.sync_copy(deg_sh.at[pl.ds(start + 4 * CHUNK, 120)],
                    ones_v.at[pl.ds(0, 120)])
    pltpu.sync_copy(ones_v.at[pl.ds(0, 120)],
                    deg_out.at[c, pl.ds(start + 4 * CHUNK, 120)])


NBUF2 = 3


def _sc_agg_body(x_hbm, src_hbm, dst_hbm, z128_hbm,
            acc_out,
            src_v, dst_v, rows_v, sem0, sem1, sem2, acc_sh):
    c = lax.axis_index("c")
    s = lax.axis_index("s")
    wid = s * NC + c
    start = lax.min(s * SLICE, N_NODES - SLICE)
    sems = (sem0, sem1, sem2)
    pltpu.sync_copy(z128_hbm, rows_v.at[0])
    for k in range(4):
        pltpu.async_copy(rows_v.at[0],
                         acc_sh.at[pl.ds(start + k * CHUNK, CHUNK)], sem0)
    pltpu.async_copy(rows_v.at[0, pl.ds(0, 120)],
                     acc_sh.at[pl.ds(start + 4 * CHUNK, 120)], sem0)
    for k in range(4):
        pltpu.make_async_copy(
            rows_v.at[0], acc_sh.at[pl.ds(start + k * CHUNK, CHUNK)],
            sem0).wait()
    pltpu.make_async_copy(
        rows_v.at[0, pl.ds(0, 120)],
        acc_sh.at[pl.ds(start + 4 * CHUNK, 120)], sem0).wait()
    plsc.subcore_barrier()

    nrows = (ROWS - wid + NW - 1) // NW

    def fire(b, i):
        j = wid + i * NW
        pltpu.sync_copy(src_hbm.at[j], src_v.at[b])
        pltpu.sync_copy(dst_hbm.at[j], dst_v.at[b])
        pltpu.async_copy(x_hbm.at[src_v.at[b]], rows_v.at[b], sems[b])

    for b in range(NBUF2):
        fire(b, b)

    def body(k, carry):
        for b in range(NBUF2):
            i = k * NBUF2 + b

            @pl.when(i < nrows)
            def _():
                pltpu.make_async_copy(x_hbm.at[src_v.at[b]], rows_v.at[b],
                                      sems[b]).wait()
                pltpu.sync_copy(rows_v.at[b], acc_sh.at[dst_v.at[b]], add=True)

                @pl.when(i + NBUF2 < nrows)
                def _():
                    fire(b, i + NBUF2)
        return carry

    lax.fori_loop(0, (nrows + NBUF2 - 1) // NBUF2, body, 0)
    plsc.subcore_barrier()
    sizes = (CHUNK, CHUNK, CHUNK, CHUNK, 120)

    def stage(b, k, n):
        pltpu.sync_copy(acc_sh.at[pl.ds(start + k * CHUNK, n)],
                        rows_v.at[b, pl.ds(0, n)])
        pltpu.async_copy(rows_v.at[b, pl.ds(0, n)],
                         acc_out.at[c, pl.ds(start + k * CHUNK, n)], sems[b])

    def drain(b, k, n):
        pltpu.make_async_copy(
            rows_v.at[b, pl.ds(0, n)],
            acc_out.at[c, pl.ds(start + k * CHUNK, n)], sems[b]).wait()

    for k, n in enumerate(sizes):
        b = k % NBUF2
        if k >= NBUF2:
            drain(b, k - NBUF2, sizes[k - NBUF2])
        stage(b, k, n)
    for k in range(max(0, len(sizes) - NBUF2), len(sizes)):
        drain(k % NBUF2, k, sizes[k])


def _make_sc_kernels(interpret=False):
    agg_deg = pl.kernel(
        _sc_agg_deg_body,
        mesh=_mesh,
        compiler_params=_sc_params,
        out_type=(
            jax.ShapeDtypeStruct((NC, N_NODES, D), jnp.float32),
            jax.ShapeDtypeStruct((NC, N_NODES, 16), jnp.float32),
        ),
        scratch_types=(
            pltpu.VMEM((NBUF, CHUNK), jnp.int32),
            pltpu.VMEM((NBUF, CHUNK), jnp.int32),
            pltpu.VMEM((NBUF, CHUNK, D), jnp.float32),
            pltpu.VMEM((CHUNK, 16), jnp.float32),
            pltpu.SemaphoreType.DMA,
            pltpu.SemaphoreType.DMA,
            pltpu.VMEM_SHARED((N_NODES, D), jnp.float32),
            pltpu.VMEM_SHARED((N_NODES, 16), jnp.float32),
        ),
        interpret=interpret,
    )
    agg = pl.kernel(
        _sc_agg_body,
        mesh=_mesh,
        compiler_params=_sc_params,
        out_type=jax.ShapeDtypeStruct((NC, N_NODES, D), jnp.float32),
        scratch_types=(
            pltpu.VMEM((NBUF2, CHUNK), jnp.int32),
            pltpu.VMEM((NBUF2, CHUNK), jnp.int32),
            pltpu.VMEM((NBUF2, CHUNK, D), jnp.float32),
            pltpu.SemaphoreType.DMA,
            pltpu.SemaphoreType.DMA,
            pltpu.SemaphoreType.DMA,
            pltpu.VMEM_SHARED((N_NODES, D), jnp.float32),
        ),
        interpret=interpret,
    )
    return agg_deg, agg


_sc_agg_deg, _sc_agg = _make_sc_kernels()


BLK = 1000
GRID = N_NODES // BLK


def _tc_layer1_body(acc_ref, deg_ref, x_ref, wl_ref, wr_ref, bl_ref,
                    h_ref, invd_ref):
    acc = acc_ref[0] + acc_ref[1]
    deg = deg_ref[0, :, 0:1] + deg_ref[1, :, 0:1]
    invd = 1.0 / jnp.maximum(deg, 1.0)
    h = jnp.dot(acc * invd, wl_ref[...], preferred_element_type=jnp.float32)
    h = h + jnp.dot(x_ref[...], wr_ref[...], preferred_element_type=jnp.float32)
    h = h + bl_ref[...]
    h_ref[...] = jnp.maximum(h, 0.0)
    invd_ref[...] = jnp.broadcast_to(invd, (BLK, 16))


def _tc_layer2_body(acc_ref, invd_ref, h1_ref, batch_ref, wl_ref, wr_ref,
                    bl_ref, wlin_ref, blin_ref, out_ref, pooled_acc, cnt_acc):
    i = pl.program_id(0)

    @pl.when(i == 0)
    def _():
        pooled_acc[...] = jnp.zeros_like(pooled_acc)
        cnt_acc[...] = jnp.zeros_like(cnt_acc)

    acc = acc_ref[0] + acc_ref[1]
    invd = invd_ref[:, 0:1]
    h2 = jnp.dot(acc * invd, wl_ref[...], preferred_element_type=jnp.float32)
    h2 = h2 + jnp.dot(h1_ref[...], wr_ref[...],
                      preferred_element_type=jnp.float32)
    h2 = jnp.maximum(h2 + bl_ref[...], 0.0)

    b = batch_ref[0]
    gids = lax.broadcasted_iota(jnp.int32, (N_GRAPHS, 1), 0)
    onehot = (gids == b).astype(jnp.float32)
    pooled_acc[...] += jnp.dot(onehot, h2, preferred_element_type=jnp.float32)
    cnt_acc[...] += jnp.broadcast_to(
        jnp.sum(onehot, axis=1, keepdims=True), (N_GRAPHS, D))

    @pl.when(i == GRID - 1)
    def _():
        pooled = pooled_acc[...] / jnp.maximum(cnt_acc[...], 1.0)
        out_ref[...] = jnp.dot(pooled, wlin_ref[...],
                               preferred_element_type=jnp.float32) + blin_ref[...]


def kernel(x, edge_index, batch, Wl1, bl1, Wr1, Wl2, bl2, Wr2, Wlin, blin):
    src2d = edge_index[0].reshape(ROWS, CHUNK)
    dst2d = edge_index[1].reshape(ROWS, CHUNK)
    ones16 = jnp.ones((CHUNK, 16), jnp.float32)
    z128 = jnp.zeros((CHUNK, D), jnp.float32)
    z16 = jnp.zeros((CHUNK, 16), jnp.float32)

    acc1, deg1 = _sc_agg_deg(x, src2d, dst2d, ones16, z128, z16)

    h1, invd16 = pl.pallas_call(
        _tc_layer1_body,
        grid=(GRID,),
        in_specs=[
            pl.BlockSpec((NC, BLK, D), lambda i: (0, i, 0)),
            pl.BlockSpec((NC, BLK, 16), lambda i: (0, i, 0)),
            pl.BlockSpec((BLK, D), lambda i: (i, 0)),
            pl.BlockSpec((D, D), lambda i: (0, 0)),
            pl.BlockSpec((D, D), lambda i: (0, 0)),
            pl.BlockSpec((1, D), lambda i: (0, 0)),
        ],
        out_specs=[
            pl.BlockSpec((BLK, D), lambda i: (i, 0)),
            pl.BlockSpec((BLK, 16), lambda i: (i, 0)),
        ],
        out_shape=[
            jax.ShapeDtypeStruct((N_NODES, D), jnp.float32),
            jax.ShapeDtypeStruct((N_NODES, 16), jnp.float32),
        ],
    )(acc1, deg1, x, Wl1.T, Wr1.T, bl1[None, :])

    acc2 = _sc_agg(h1, src2d, dst2d, z128)

    wlin_pad = jnp.pad(Wlin.T, ((0, 0), (0, D - D_OUT)))
    blin_pad = jnp.pad(blin[None, :], ((0, 0), (0, D - D_OUT)))
    batch3d = batch.reshape(GRID, 1, BLK)

    out_pad = pl.pallas_call(
        _tc_layer2_body,
        grid=(GRID,),
        in_specs=[
            pl.BlockSpec((NC, BLK, D), lambda i: (0, i, 0)),
            pl.BlockSpec((BLK, 16), lambda i: (i, 0)),
            pl.BlockSpec((BLK, D), lambda i: (i, 0)),
            pl.BlockSpec((1, 1, BLK), lambda i: (i, 0, 0)),
            pl.BlockSpec((D, D), lambda i: (0, 0)),
            pl.BlockSpec((D, D), lambda i: (0, 0)),
            pl.BlockSpec((1, D), lambda i: (0, 0)),
            pl.BlockSpec((D, D), lambda i: (0, 0)),
            pl.BlockSpec((1, D), lambda i: (0, 0)),
        ],
        out_specs=pl.BlockSpec((N_GRAPHS, D), lambda i: (0, 0)),
        out_shape=jax.ShapeDtypeStruct((N_GRAPHS, D), jnp.float32),
        scratch_shapes=[
            pltpu.VMEM((N_GRAPHS, D), jnp.float32),
            pltpu.VMEM((N_GRAPHS, D), jnp.float32),
        ],
    )(acc2, invd16, h1, batch3d, Wl2.T, Wr2.T, bl2[None, :],
      wlin_pad, blin_pad)

    return out_pad[:, :D_OUT]


# speedup vs baseline: 14.5926x; 1.3951x over previous
import functools

import jax
import jax.numpy as jnp
from jax import lax
from jax.experimental import pallas as pl
from jax.experimental.pallas import tpu as pltpu
from jax.experimental.pallas import tpu_sc as plsc

N_NODES = 10000
N_EDGES = 320000
N_GRAPHS = 64
D = 128
D_OUT = 2

CHUNK = 128
ROWS = N_EDGES // CHUNK
NC = 2
NS = 16
NW = NC * NS
SLICE = 632

_mesh = plsc.VectorSubcoreMesh(core_axis_name="c", subcore_axis_name="s")
_sc_params = pltpu.CompilerParams(use_tc_tiling_on_sc=False)


NBUF = 2


def _sc_agg_deg_body(x_hbm, ei_hbm, ones_hbm, z128_hbm, z16_hbm,
                acc_out, deg_out,
                idx_v, rows_v, ones_v,
                sem0, sem1, isem0, isem1, acc_sh, deg_sh):
    c = lax.axis_index("c")
    s = lax.axis_index("s")
    wid = s * NC + c
    start = lax.min(s * SLICE, N_NODES - SLICE)
    pltpu.sync_copy(z128_hbm, rows_v.at[0])
    pltpu.sync_copy(z16_hbm, ones_v)
    for k in range(4):
        pltpu.async_copy(rows_v.at[0],
                         acc_sh.at[pl.ds(start + k * CHUNK, CHUNK)], sem0)
        pltpu.async_copy(ones_v,
                         deg_sh.at[pl.ds(start + k * CHUNK, CHUNK)], sem1)
    pltpu.async_copy(rows_v.at[0, pl.ds(0, 120)],
                     acc_sh.at[pl.ds(start + 4 * CHUNK, 120)], sem0)
    pltpu.async_copy(ones_v.at[pl.ds(0, 120)],
                     deg_sh.at[pl.ds(start + 4 * CHUNK, 120)], sem1)
    for k in range(4):
        pltpu.make_async_copy(
            rows_v.at[0], acc_sh.at[pl.ds(start + k * CHUNK, CHUNK)],
            sem0).wait()
        pltpu.make_async_copy(
            ones_v, deg_sh.at[pl.ds(start + k * CHUNK, CHUNK)], sem1).wait()
    pltpu.make_async_copy(
        rows_v.at[0, pl.ds(0, 120)],
        acc_sh.at[pl.ds(start + 4 * CHUNK, 120)], sem0).wait()
    pltpu.make_async_copy(
        ones_v.at[pl.ds(0, 120)],
        deg_sh.at[pl.ds(start + 4 * CHUNK, 120)], sem1).wait()
    pltpu.sync_copy(ones_hbm, ones_v)
    plsc.subcore_barrier()

    nrows = (ROWS - wid + NW - 1) // NW
    sems = (sem0, sem1)
    isems = (isem0, isem1)

    def fire_idx(b, q, i):
        j = wid + i * NW
        pltpu.async_copy(ei_hbm.at[j], idx_v.at[b, q], isems[b])

    def wait_idx(b, q):
        pltpu.make_async_copy(ei_hbm.at[0], idx_v.at[b, q], isems[b]).wait()

    def fire_gather(b, q):
        pltpu.async_copy(x_hbm.at[idx_v.at[b, q, 0]], rows_v.at[b], sems[b])

    for b in range(NBUF):
        fire_idx(b, 0, b)
        wait_idx(b, 0)
        fire_gather(b, 0)

    def body(k2, carry):
        for q in range(2):
            for b in range(NBUF):
                i = k2 * (2 * NBUF) + q * NBUF + b

                @pl.when(i < nrows)
                def _():
                    pltpu.make_async_copy(x_hbm.at[idx_v.at[b, q, 0]],
                                          rows_v.at[b], sems[b]).wait()

                    @pl.when(i + NBUF < nrows)
                    def _():
                        fire_idx(b, 1 - q, i + NBUF)

                    pltpu.sync_copy(rows_v.at[b],
                                    acc_sh.at[idx_v.at[b, q, 1]], add=True)
                    pltpu.sync_copy(ones_v,
                                    deg_sh.at[idx_v.at[b, q, 1]], add=True)

                    @pl.when(i + NBUF < nrows)
                    def _():
                        wait_idx(b, 1 - q)
                        fire_gather(b, 1 - q)
        return carry

    lax.fori_loop(0, (nrows + 2 * NBUF - 1) // (2 * NBUF), body, 0)
    plsc.subcore_barrier()
    sizes = (CHUNK, CHUNK, CHUNK, CHUNK, 120)

    def stage(b, k, n):
        pltpu.sync_copy(acc_sh.at[pl.ds(start + k * CHUNK, n)],
                        rows_v.at[b, pl.ds(0, n)])
        pltpu.async_copy(rows_v.at[b, pl.ds(0, n)],
                         acc_out.at[c, pl.ds(start + k * CHUNK, n)], sems[b])

    def drain(b, k, n):
        pltpu.make_async_copy(
            rows_v.at[b, pl.ds(0, n)],
            acc_out.at[c, pl.ds(start + k * CHUNK, n)], sems[b]).wait()

    for k, n in enumerate(sizes):
        b = k % NBUF
        if k >= NBUF:
            drain(b, k - NBUF, sizes[k - NBUF])
        stage(b, k, n)
    for k in range(max(0, len(sizes) - NBUF), len(sizes)):
        drain(k % NBUF, k, sizes[k])
    for k in range(4):
        pltpu.sync_copy(deg_sh.at[pl.ds(start + k * CHUNK, CHUNK)], ones_v)
        pltpu.sync_copy(ones_v, deg_out.at[c, pl.ds(start + k * CHUNK, CHUNK)])
    pltpu.sync_copy(deg_sh.at[pl.ds(start + 4 * CHUNK, 120)],
                    ones_v.at[pl.ds(0, 120)])
    pltpu.sync_copy(ones_v.at[pl.ds(0, 120)],
                    deg_out.at[c, pl.ds(start + 4 * CHUNK, 120)])


NBUF2 = 3


def _sc_agg_body(x_hbm, ei_hbm, z128_hbm,
            acc_out,
            idx_v, rows_v, sem0, sem1, sem2, isem0, isem1, isem2, acc_sh):
    c = lax.axis_index("c")
    s = lax.axis_index("s")
    wid = s * NC + c
    start = lax.min(s * SLICE, N_NODES - SLICE)
    sems = (sem0, sem1, sem2)
    pltpu.sync_copy(z128_hbm, rows_v.at[0])
    for k in range(4):
        pltpu.async_copy(rows_v.at[0],
                         acc_sh.at[pl.ds(start + k * CHUNK, CHUNK)], sem0)
    pltpu.async_copy(rows_v.at[0, pl.ds(0, 120)],
                     acc_sh.at[pl.ds(start + 4 * CHUNK, 120)], sem0)
    for k in range(4):
        pltpu.make_async_copy(
            rows_v.at[0], acc_sh.at[pl.ds(start + k * CHUNK, CHUNK)],
            sem0).wait()
    pltpu.make_async_copy(
        rows_v.at[0, pl.ds(0, 120)],
        acc_sh.at[pl.ds(start + 4 * CHUNK, 120)], sem0).wait()
    plsc.subcore_barrier()

    nrows = (ROWS - wid + NW - 1) // NW
    isems = (isem0, isem1, isem2)

    def fire_idx(b, q, i):
        j = wid + i * NW
        pltpu.async_copy(ei_hbm.at[j], idx_v.at[b, q], isems[b])

    def wait_idx(b, q):
        pltpu.make_async_copy(ei_hbm.at[0], idx_v.at[b, q], isems[b]).wait()

    def fire_gather(b, q):
        pltpu.async_copy(x_hbm.at[idx_v.at[b, q, 0]], rows_v.at[b], sems[b])

    for b in range(NBUF2):
        fire_idx(b, 0, b)
        wait_idx(b, 0)
        fire_gather(b, 0)

    def body(k2, carry):
        for q in range(2):
            for b in range(NBUF2):
                i = k2 * (2 * NBUF2) + q * NBUF2 + b

                @pl.when(i < nrows)
                def _():
                    pltpu.make_async_copy(x_hbm.at[idx_v.at[b, q, 0]],
                                          rows_v.at[b], sems[b]).wait()

                    @pl.when(i + NBUF2 < nrows)
                    def _():
                        fire_idx(b, 1 - q, i + NBUF2)

                    pltpu.sync_copy(rows_v.at[b],
                                    acc_sh.at[idx_v.at[b, q, 1]], add=True)

                    @pl.when(i + NBUF2 < nrows)
                    def _():
                        wait_idx(b, 1 - q)
                        fire_gather(b, 1 - q)
        return carry

    lax.fori_loop(0, (nrows + 2 * NBUF2 - 1) // (2 * NBUF2), body, 0)
    plsc.subcore_barrier()
    sizes = (CHUNK, CHUNK, CHUNK, CHUNK, 120)

    def stage(b, k, n):
        pltpu.sync_copy(acc_sh.at[pl.ds(start + k * CHUNK, n)],
                        rows_v.at[b, pl.ds(0, n)])
        pltpu.async_copy(rows_v.at[b, pl.ds(0, n)],
                         acc_out.at[c, pl.ds(start + k * CHUNK, n)], sems[b])

    def drain(b, k, n):
        pltpu.make_async_copy(
            rows_v.at[b, pl.ds(0, n)],
            acc_out.at[c, pl.ds(start + k * CHUNK, n)], sems[b]).wait()

    for k, n in enumerate(sizes):
        b = k % NBUF2
        if k >= NBUF2:
            drain(b, k - NBUF2, sizes[k - NBUF2])
        stage(b, k, n)
    for k in range(max(0, len(sizes) - NBUF2), len(sizes)):
        drain(k % NBUF2, k, sizes[k])


def _make_sc_kernels(interpret=False):
    agg_deg = pl.kernel(
        _sc_agg_deg_body,
        mesh=_mesh,
        compiler_params=_sc_params,
        out_type=(
            jax.ShapeDtypeStruct((NC, N_NODES, D), jnp.float32),
            jax.ShapeDtypeStruct((NC, N_NODES, 16), jnp.float32),
        ),
        scratch_types=(
            pltpu.VMEM((NBUF, 2, 2, CHUNK), jnp.int32),
            pltpu.VMEM((NBUF, CHUNK, D), jnp.float32),
            pltpu.VMEM((CHUNK, 16), jnp.float32),
            pltpu.SemaphoreType.DMA,
            pltpu.SemaphoreType.DMA,
            pltpu.SemaphoreType.DMA,
            pltpu.SemaphoreType.DMA,
            pltpu.VMEM_SHARED((N_NODES, D), jnp.float32),
            pltpu.VMEM_SHARED((N_NODES, 16), jnp.float32),
        ),
        interpret=interpret,
    )
    agg = pl.kernel(
        _sc_agg_body,
        mesh=_mesh,
        compiler_params=_sc_params,
        out_type=jax.ShapeDtypeStruct((NC, N_NODES, D), jnp.float32),
        scratch_types=(
            pltpu.VMEM((NBUF2, 2, 2, CHUNK), jnp.int32),
            pltpu.VMEM((NBUF2, CHUNK, D), jnp.float32),
            pltpu.SemaphoreType.DMA,
            pltpu.SemaphoreType.DMA,
            pltpu.SemaphoreType.DMA,
            pltpu.SemaphoreType.DMA,
            pltpu.SemaphoreType.DMA,
            pltpu.SemaphoreType.DMA,
            pltpu.VMEM_SHARED((N_NODES, D), jnp.float32),
        ),
        interpret=interpret,
    )
    return agg_deg, agg


_sc_agg_deg, _sc_agg = _make_sc_kernels()


BLK = 1000
GRID = N_NODES // BLK


def _tc_layer1_body(acc_ref, deg_ref, x_ref, wl_ref, wr_ref, bl_ref,
                    h_ref, invd_ref):
    acc = acc_ref[0] + acc_ref[1]
    deg = deg_ref[0, :, 0:1] + deg_ref[1, :, 0:1]
    invd = 1.0 / jnp.maximum(deg, 1.0)
    h = jnp.dot(acc * invd, wl_ref[...], preferred_element_type=jnp.float32)
    h = h + jnp.dot(x_ref[...], wr_ref[...], preferred_element_type=jnp.float32)
    h = h + bl_ref[...]
    h_ref[...] = jnp.maximum(h, 0.0)
    invd_ref[...] = jnp.broadcast_to(invd, (BLK, 16))


def _tc_layer2_body(acc_ref, invd_ref, h1_ref, batch_ref, wl_ref, wr_ref,
                    bl_ref, wlin_ref, blin_ref, out_ref, pooled_acc, cnt_acc):
    i = pl.program_id(0)

    @pl.when(i == 0)
    def _():
        pooled_acc[...] = jnp.zeros_like(pooled_acc)
        cnt_acc[...] = jnp.zeros_like(cnt_acc)

    acc = acc_ref[0] + acc_ref[1]
    invd = invd_ref[:, 0:1]
    h2 = jnp.dot(acc * invd, wl_ref[...], preferred_element_type=jnp.float32)
    h2 = h2 + jnp.dot(h1_ref[...], wr_ref[...],
                      preferred_element_type=jnp.float32)
    h2 = jnp.maximum(h2 + bl_ref[...], 0.0)

    b = batch_ref[0]
    gids = lax.broadcasted_iota(jnp.int32, (N_GRAPHS, 1), 0)
    onehot = (gids == b).astype(jnp.float32)
    pooled_acc[...] += jnp.dot(onehot, h2, preferred_element_type=jnp.float32)
    cnt_acc[...] += jnp.broadcast_to(
        jnp.sum(onehot, axis=1, keepdims=True), (N_GRAPHS, D))

    @pl.when(i == GRID - 1)
    def _():
        pooled = pooled_acc[...] / jnp.maximum(cnt_acc[...], 1.0)
        out_ref[...] = jnp.dot(pooled, wlin_ref[...],
                               preferred_element_type=jnp.float32) + blin_ref[...]


def kernel(x, edge_index, batch, Wl1, bl1, Wr1, Wl2, bl2, Wr2, Wlin, blin):
    src2d = edge_index[0].reshape(ROWS, CHUNK)
    dst2d = edge_index[1].reshape(ROWS, CHUNK)
    ei2 = jnp.stack((src2d, dst2d), axis=1)
    ones16 = jnp.ones((CHUNK, 16), jnp.float32)
    z128 = jnp.zeros((CHUNK, D), jnp.float32)
    z16 = jnp.zeros((CHUNK, 16), jnp.float32)

    acc1, deg1 = _sc_agg_deg(x, ei2, ones16, z128, z16)

    h1, invd16 = pl.pallas_call(
        _tc_layer1_body,
        grid=(GRID,),
        in_specs=[
            pl.BlockSpec((NC, BLK, D), lambda i: (0, i, 0)),
            pl.BlockSpec((NC, BLK, 16), lambda i: (0, i, 0)),
            pl.BlockSpec((BLK, D), lambda i: (i, 0)),
            pl.BlockSpec((D, D), lambda i: (0, 0)),
            pl.BlockSpec((D, D), lambda i: (0, 0)),
            pl.BlockSpec((1, D), lambda i: (0, 0)),
        ],
        out_specs=[
            pl.BlockSpec((BLK, D), lambda i: (i, 0)),
            pl.BlockSpec((BLK, 16), lambda i: (i, 0)),
        ],
        out_shape=[
            jax.ShapeDtypeStruct((N_NODES, D), jnp.float32),
            jax.ShapeDtypeStruct((N_NODES, 16), jnp.float32),
        ],
    )(acc1, deg1, x, Wl1.T, Wr1.T, bl1[None, :])

    acc2 = _sc_agg(h1, ei2, z128)

    wlin_pad = jnp.pad(Wlin.T, ((0, 0), (0, D - D_OUT)))
    blin_pad = jnp.pad(blin[None, :], ((0, 0), (0, D - D_OUT)))
    batch3d = batch.reshape(GRID, 1, BLK)

    out_pad = pl.pallas_call(
        _tc_layer2_body,
        grid=(GRID,),
        in_specs=[
            pl.BlockSpec((NC, BLK, D), lambda i: (0, i, 0)),
            pl.BlockSpec((BLK, 16), lambda i: (i, 0)),
            pl.BlockSpec((BLK, D), lambda i: (i, 0)),
            pl.BlockSpec((1, 1, BLK), lambda i: (i, 0, 0)),
            pl.BlockSpec((D, D), lambda i: (0, 0)),
            pl.BlockSpec((D, D), lambda i: (0, 0)),
            pl.BlockSpec((1, D), lambda i: (0, 0)),
            pl.BlockSpec((D, D), lambda i: (0, 0)),
            pl.BlockSpec((1, D), lambda i: (0, 0)),
        ],
        out_specs=pl.BlockSpec((N_GRAPHS, D), lambda i: (0, 0)),
        out_shape=jax.ShapeDtypeStruct((N_GRAPHS, D), jnp.float32),
        scratch_shapes=[
            pltpu.VMEM((N_GRAPHS, D), jnp.float32),
            pltpu.VMEM((N_GRAPHS, D), jnp.float32),
        ],
    )(acc2, invd16, h1, batch3d, Wl2.T, Wr2.T, bl2[None, :],
      wlin_pad, blin_pad)

    return out_pad[:, :D_OUT]


# speedup vs baseline: 14.6124x; 1.0014x over previous
import functools

import jax
import jax.numpy as jnp
from jax import lax
from jax.experimental import pallas as pl
from jax.experimental.pallas import tpu as pltpu
from jax.experimental.pallas import tpu_sc as plsc

N_NODES = 10000
N_EDGES = 320000
N_GRAPHS = 64
D = 128
D_OUT = 2

CHUNK = 128
ROWS = N_EDGES // CHUNK
NC = 2
NS = 16
NW = NC * NS
SLICE = 632

_mesh = plsc.VectorSubcoreMesh(core_axis_name="c", subcore_axis_name="s")
_sc_params = pltpu.CompilerParams(use_tc_tiling_on_sc=False)


NBUF = 2


def _sc_agg_deg_body(x_hbm, ei_hbm, ones_hbm, z128_hbm, z16_hbm,
                acc_out, deg_out,
                idx_v, rows_v, ones_v,
                sem0, sem1, isem0, isem1, acc_sh, deg_sh):
    c = lax.axis_index("c")
    s = lax.axis_index("s")
    wid = s * NC + c
    start = lax.min(s * SLICE, N_NODES - SLICE)
    pltpu.sync_copy(z128_hbm, rows_v.at[0])
    pltpu.sync_copy(z16_hbm, ones_v)
    for k in range(4):
        pltpu.async_copy(rows_v.at[0],
                         acc_sh.at[pl.ds(start + k * CHUNK, CHUNK)], sem0)
        pltpu.async_copy(ones_v,
                         deg_sh.at[pl.ds(start + k * CHUNK, CHUNK)], sem1)
    pltpu.async_copy(rows_v.at[0, pl.ds(0, 120)],
                     acc_sh.at[pl.ds(start + 4 * CHUNK, 120)], sem0)
    pltpu.async_copy(ones_v.at[pl.ds(0, 120)],
                     deg_sh.at[pl.ds(start + 4 * CHUNK, 120)], sem1)
    for k in range(4):
        pltpu.make_async_copy(
            rows_v.at[0], acc_sh.at[pl.ds(start + k * CHUNK, CHUNK)],
            sem0).wait()
        pltpu.make_async_copy(
            ones_v, deg_sh.at[pl.ds(start + k * CHUNK, CHUNK)], sem1).wait()
    pltpu.make_async_copy(
        rows_v.at[0, pl.ds(0, 120)],
        acc_sh.at[pl.ds(start + 4 * CHUNK, 120)], sem0).wait()
    pltpu.make_async_copy(
        ones_v.at[pl.ds(0, 120)],
        deg_sh.at[pl.ds(start + 4 * CHUNK, 120)], sem1).wait()
    pltpu.sync_copy(ones_hbm, ones_v)
    plsc.subcore_barrier()

    nrows = (ROWS - wid + NW - 1) // NW
    sems = (sem0, sem1)
    isems = (isem0, isem1)

    def fire_idx(b, q, i):
        j = wid + i * NW
        pltpu.async_copy(ei_hbm.at[j], idx_v.at[b, q], isems[b])

    def wait_idx(b, q):
        pltpu.make_async_copy(ei_hbm.at[0], idx_v.at[b, q], isems[b]).wait()

    def fire_gather(b, q):
        pltpu.async_copy(x_hbm.at[idx_v.at[b, q, 0]], rows_v.at[b], sems[b])

    for b in range(NBUF):
        fire_idx(b, 0, b)
        wait_idx(b, 0)
        fire_gather(b, 0)

    def body(k2, carry):
        for q in range(2):
            for b in range(NBUF):
                i = k2 * (2 * NBUF) + q * NBUF + b

                @pl.when(i < nrows)
                def _():
                    pltpu.make_async_copy(x_hbm.at[idx_v.at[b, q, 0]],
                                          rows_v.at[b], sems[b]).wait()

                    @pl.when(i + NBUF < nrows)
                    def _():
                        fire_idx(b, 1 - q, i + NBUF)

                    pltpu.sync_copy(rows_v.at[b],
                                    acc_sh.at[idx_v.at[b, q, 1]], add=True)
                    pltpu.sync_copy(ones_v,
                                    deg_sh.at[idx_v.at[b, q, 1]], add=True)

                    @pl.when(i + NBUF < nrows)
                    def _():
                        wait_idx(b, 1 - q)
                        fire_gather(b, 1 - q)
        return carry

    lax.fori_loop(0, (nrows + 2 * NBUF - 1) // (2 * NBUF), body, 0)
    plsc.subcore_barrier()
    sizes = (CHUNK, CHUNK, CHUNK, CHUNK, 120)

    def stage(b, k, n):
        pltpu.sync_copy(acc_sh.at[pl.ds(start + k * CHUNK, n)],
                        rows_v.at[b, pl.ds(0, n)])
        pltpu.async_copy(rows_v.at[b, pl.ds(0, n)],
                         acc_out.at[c, pl.ds(start + k * CHUNK, n)], sems[b])

    def drain(b, k, n):
        pltpu.make_async_copy(
            rows_v.at[b, pl.ds(0, n)],
            acc_out.at[c, pl.ds(start + k * CHUNK, n)], sems[b]).wait()

    for k, n in enumerate(sizes):
        b = k % NBUF
        if k >= NBUF:
            drain(b, k - NBUF, sizes[k - NBUF])
        stage(b, k, n)
    for k in range(max(0, len(sizes) - NBUF), len(sizes)):
        drain(k % NBUF, k, sizes[k])
    for k in range(4):
        pltpu.sync_copy(deg_sh.at[pl.ds(start + k * CHUNK, CHUNK)], ones_v)
        pltpu.sync_copy(ones_v, deg_out.at[c, pl.ds(start + k * CHUNK, CHUNK)])
    pltpu.sync_copy(deg_sh.at[pl.ds(start + 4 * CHUNK, 120)],
                    ones_v.at[pl.ds(0, 120)])
    pltpu.sync_copy(ones_v.at[pl.ds(0, 120)],
                    deg_out.at[c, pl.ds(start + 4 * CHUNK, 120)])


NBUF2 = 3


def _sc_agg_body(x_hbm, ei_hbm, z128_hbm,
            acc_out,
            idx_v, rows_v, sem0, sem1, sem2, isem0, isem1, isem2, acc_sh):
    c = lax.axis_index("c")
    s = lax.axis_index("s")
    wid = s * NC + c
    start = lax.min(s * SLICE, N_NODES - SLICE)
    sems = (sem0, sem1, sem2)
    pltpu.sync_copy(z128_hbm, rows_v.at[0])
    for k in range(4):
        pltpu.async_copy(rows_v.at[0],
                         acc_sh.at[pl.ds(start + k * CHUNK, CHUNK)], sem0)
    pltpu.async_copy(rows_v.at[0, pl.ds(0, 120)],
                     acc_sh.at[pl.ds(start + 4 * CHUNK, 120)], sem0)
    for k in range(4):
        pltpu.make_async_copy(
            rows_v.at[0], acc_sh.at[pl.ds(start + k * CHUNK, CHUNK)],
            sem0).wait()
    pltpu.make_async_copy(
        rows_v.at[0, pl.ds(0, 120)],
        acc_sh.at[pl.ds(start + 4 * CHUNK, 120)], sem0).wait()
    plsc.subcore_barrier()

    nrows = (ROWS - wid + NW - 1) // NW
    isems = (isem0, isem1, isem2)

    def fire_idx(b, q, i):
        j = wid + i * NW
        pltpu.async_copy(ei_hbm.at[j], idx_v.at[b, q], isems[b])

    def wait_idx(b, q):
        pltpu.make_async_copy(ei_hbm.at[0], idx_v.at[b, q], isems[b]).wait()

    def fire_gather(b, q):
        pltpu.async_copy(x_hbm.at[idx_v.at[b, q, 0]], rows_v.at[b], sems[b])

    for b in range(NBUF2):
        fire_idx(b, 0, b)
        wait_idx(b, 0)
        fire_gather(b, 0)

    def body(k2, carry):
        for q in range(2):
            for b in range(NBUF2):
                i = k2 * (2 * NBUF2) + q * NBUF2 + b

                @pl.when(i < nrows)
                def _():
                    pltpu.make_async_copy(x_hbm.at[idx_v.at[b, q, 0]],
                                          rows_v.at[b], sems[b]).wait()

                    @pl.when(i + NBUF2 < nrows)
                    def _():
                        fire_idx(b, 1 - q, i + NBUF2)

                    pltpu.sync_copy(rows_v.at[b],
                                    acc_sh.at[idx_v.at[b, q, 1]], add=True)

                    @pl.when(i + NBUF2 < nrows)
                    def _():
                        wait_idx(b, 1 - q)
                        fire_gather(b, 1 - q)
        return carry

    lax.fori_loop(0, (nrows + 2 * NBUF2 - 1) // (2 * NBUF2), body, 0)
    plsc.subcore_barrier()
    sizes = (CHUNK, CHUNK, CHUNK, CHUNK, 120)

    def stage(b, k, n):
        pltpu.sync_copy(acc_sh.at[pl.ds(start + k * CHUNK, n)],
                        rows_v.at[b, pl.ds(0, n)])
        pltpu.async_copy(rows_v.at[b, pl.ds(0, n)],
                         acc_out.at[c, pl.ds(start + k * CHUNK, n)], sems[b])

    def drain(b, k, n):
        pltpu.make_async_copy(
            rows_v.at[b, pl.ds(0, n)],
            acc_out.at[c, pl.ds(start + k * CHUNK, n)], sems[b]).wait()

    for k, n in enumerate(sizes):
        b = k % NBUF2
        if k >= NBUF2:
            drain(b, k - NBUF2, sizes[k - NBUF2])
        stage(b, k, n)
    for k in range(max(0, len(sizes) - NBUF2), len(sizes)):
        drain(k % NBUF2, k, sizes[k])


def _make_sc_kernels():
    agg_deg = pl.kernel(
        _sc_agg_deg_body,
        mesh=_mesh,
        compiler_params=_sc_params,
        out_type=(
            jax.ShapeDtypeStruct((NC, N_NODES, D), jnp.float32),
            jax.ShapeDtypeStruct((NC, N_NODES, 16), jnp.float32),
        ),
        scratch_types=(
            pltpu.VMEM((NBUF, 2, 2, CHUNK), jnp.int32),
            pltpu.VMEM((NBUF, CHUNK, D), jnp.float32),
            pltpu.VMEM((CHUNK, 16), jnp.float32),
            pltpu.SemaphoreType.DMA,
            pltpu.SemaphoreType.DMA,
            pltpu.SemaphoreType.DMA,
            pltpu.SemaphoreType.DMA,
            pltpu.VMEM_SHARED((N_NODES, D), jnp.float32),
            pltpu.VMEM_SHARED((N_NODES, 16), jnp.float32),
        ),
    )
    agg = pl.kernel(
        _sc_agg_body,
        mesh=_mesh,
        compiler_params=_sc_params,
        out_type=jax.ShapeDtypeStruct((NC, N_NODES, D), jnp.float32),
        scratch_types=(
            pltpu.VMEM((NBUF2, 2, 2, CHUNK), jnp.int32),
            pltpu.VMEM((NBUF2, CHUNK, D), jnp.float32),
            pltpu.SemaphoreType.DMA,
            pltpu.SemaphoreType.DMA,
            pltpu.SemaphoreType.DMA,
            pltpu.SemaphoreType.DMA,
            pltpu.SemaphoreType.DMA,
            pltpu.SemaphoreType.DMA,
            pltpu.VMEM_SHARED((N_NODES, D), jnp.float32),
        ),
    )
    return agg_deg, agg


_sc_agg_deg, _sc_agg = _make_sc_kernels()


BLK = 1000
GRID = N_NODES // BLK


def _tc_layer1_body(acc_ref, deg_ref, x_ref, wl_ref, wr_ref, bl_ref,
                    h_ref, invd_ref):
    acc = acc_ref[0] + acc_ref[1]
    deg = deg_ref[0, :, 0:1] + deg_ref[1, :, 0:1]
    invd = 1.0 / jnp.maximum(deg, 1.0)
    h = jnp.dot(acc * invd, wl_ref[...], preferred_element_type=jnp.float32)
    h = h + jnp.dot(x_ref[...], wr_ref[...], preferred_element_type=jnp.float32)
    h = h + bl_ref[...]
    h_ref[...] = jnp.maximum(h, 0.0)
    invd_ref[...] = jnp.broadcast_to(invd, (BLK, 16))


def _tc_layer2_body(acc_ref, invd_ref, h1_ref, batch_ref, wl_ref, wr_ref,
                    bl_ref, wlin_ref, blin_ref, out_ref, pooled_acc, cnt_acc):
    i = pl.program_id(0)

    @pl.when(i == 0)
    def _():
        pooled_acc[...] = jnp.zeros_like(pooled_acc)
        cnt_acc[...] = jnp.zeros_like(cnt_acc)

    acc = acc_ref[0] + acc_ref[1]
    invd = invd_ref[:, 0:1]
    h2 = jnp.dot(acc * invd, wl_ref[...], preferred_element_type=jnp.float32)
    h2 = h2 + jnp.dot(h1_ref[...], wr_ref[...],
                      preferred_element_type=jnp.float32)
    h2 = jnp.maximum(h2 + bl_ref[...], 0.0)

    b = batch_ref[0]
    gids = lax.broadcasted_iota(jnp.int32, (N_GRAPHS, 1), 0)
    onehot = (gids == b).astype(jnp.float32)
    pooled_acc[...] += jnp.dot(onehot, h2, preferred_element_type=jnp.float32)
    cnt_acc[...] += jnp.broadcast_to(
        jnp.sum(onehot, axis=1, keepdims=True), (N_GRAPHS, D))

    @pl.when(i == GRID - 1)
    def _():
        pooled = pooled_acc[...] / jnp.maximum(cnt_acc[...], 1.0)
        out_ref[...] = jnp.dot(pooled, wlin_ref[...],
                               preferred_element_type=jnp.float32) + blin_ref[...]


def kernel(x, edge_index, batch, Wl1, bl1, Wr1, Wl2, bl2, Wr2, Wlin, blin):
    src2d = edge_index[0].reshape(ROWS, CHUNK)
    dst2d = edge_index[1].reshape(ROWS, CHUNK)
    ei2 = jnp.stack((src2d, dst2d), axis=1)
    ones16 = jnp.ones((CHUNK, 16), jnp.float32)
    z128 = jnp.zeros((CHUNK, D), jnp.float32)
    z16 = jnp.zeros((CHUNK, 16), jnp.float32)

    acc1, deg1 = _sc_agg_deg(x, ei2, ones16, z128, z16)

    h1, invd16 = pl.pallas_call(
        _tc_layer1_body,
        grid=(GRID,),
        in_specs=[
            pl.BlockSpec((NC, BLK, D), lambda i: (0, i, 0)),
            pl.BlockSpec((NC, BLK, 16), lambda i: (0, i, 0)),
            pl.BlockSpec((BLK, D), lambda i: (i, 0)),
            pl.BlockSpec((D, D), lambda i: (0, 0)),
            pl.BlockSpec((D, D), lambda i: (0, 0)),
            pl.BlockSpec((1, D), lambda i: (0, 0)),
        ],
        out_specs=[
            pl.BlockSpec((BLK, D), lambda i: (i, 0)),
            pl.BlockSpec((BLK, 16), lambda i: (i, 0)),
        ],
        out_shape=[
            jax.ShapeDtypeStruct((N_NODES, D), jnp.float32),
            jax.ShapeDtypeStruct((N_NODES, 16), jnp.float32),
        ],
    )(acc1, deg1, x, Wl1.T, Wr1.T, bl1[None, :])

    acc2 = _sc_agg(h1, ei2, z128)

    wlin_pad = jnp.pad(Wlin.T, ((0, 0), (0, D - D_OUT)))
    blin_pad = jnp.pad(blin[None, :], ((0, 0), (0, D - D_OUT)))
    batch3d = batch.reshape(GRID, 1, BLK)

    out_pad = pl.pallas_call(
        _tc_layer2_body,
        grid=(GRID,),
        in_specs=[
            pl.BlockSpec((NC, BLK, D), lambda i: (0, i, 0)),
            pl.BlockSpec((BLK, 16), lambda i: (i, 0)),
            pl.BlockSpec((BLK, D), lambda i: (i, 0)),
            pl.BlockSpec((1, 1, BLK), lambda i: (i, 0, 0)),
            pl.BlockSpec((D, D), lambda i: (0, 0)),
            pl.BlockSpec((D, D), lambda i: (0, 0)),
            pl.BlockSpec((1, D), lambda i: (0, 0)),
            pl.BlockSpec((D, D), lambda i: (0, 0)),
            pl.BlockSpec((1, D), lambda i: (0, 0)),
        ],
        out_specs=pl.BlockSpec((N_GRAPHS, D), lambda i: (0, 0)),
        out_shape=jax.ShapeDtypeStruct((N_GRAPHS, D), jnp.float32),
        scratch_shapes=[
            pltpu.VMEM((N_GRAPHS, D), jnp.float32),
            pltpu.VMEM((N_GRAPHS, D), jnp.float32),
        ],
    )(acc2, invd16, h1, batch3d, Wl2.T, Wr2.T, bl2[None, :],
      wlin_pad, blin_pad)

    return out_pad[:, :D_OUT]
